# baseline (speedup 1.0000x reference)
import functools

import jax
import jax.numpy as jnp
from jax import lax
from jax.experimental import pallas as pl
from jax.experimental.pallas import tpu as pltpu
from jax.experimental.pallas import tpu_sc as plsc

NC = 2
NS = 16
NW = NC * NS
DPAD = 1024


@functools.cache
def _make_loss_tc(n_blocks, blk, v):
    def body(x_ref, y_ref, t_ref, o_ref, c_ref):
        i = pl.program_id(0)

        @pl.when(i == 0)
        def _init():
            c_ref[...] = jnp.zeros_like(c_ref)

        xb = x_ref[0, 0, :]
        yb = y_ref[0, 0, :]
        ids = lax.broadcasted_iota(jnp.int32, (blk, v), 1)
        ohx = (xb[:, None] == ids).astype(jnp.bfloat16)
        ohy = (yb[:, None] == ids).astype(jnp.bfloat16)
        c_ref[...] += lax.dot_general(
            ohx, ohy, (((0,), (0,)), ((), ())),
            preferred_element_type=jnp.float32)

        @pl.when(i == n_blocks - 1)
        def _finish():
            t = t_ref[...]
            m = jnp.max(t, axis=1)
            lse = m + jnp.log(jnp.sum(jnp.exp(t - m[:, None]), axis=1))
            c = c_ref[...]
            hist_x = jnp.sum(c, axis=1)
            raw = jnp.sum(hist_x * lse) - jnp.sum(c * t)
            o_ref[...] = jnp.full((1, 1), raw, jnp.float32)

    return pl.pallas_call(
        body,
        grid=(n_blocks,),
        in_specs=[
            pl.BlockSpec((1, 1, blk), lambda i: (i, 0, 0)),
            pl.BlockSpec((1, 1, blk), lambda i: (i, 0, 0)),
            pl.BlockSpec((v, v), lambda i: (0, 0)),
        ],
        out_specs=pl.BlockSpec((1, 1), lambda i: (0, 0)),
        out_shape=jax.ShapeDtypeStruct((1, 1), jnp.float32),
        scratch_shapes=[pltpu.VMEM((v, v), jnp.float32)],
    )


@functools.cache
def _make_patch_tc(n, d, blk):
    d_full = (d // 128) * 128
    de = d - d_full
    n_blk = n // blk
    assert n % blk == 0

    def body(alias_ref, edge_h, o_ref, r0, r1, w0, w1, sr0, sr1, sw0, sw1):
        del alias_ref
        rbufs = (r0, r1)
        wbufs = (w0, w1)
        srs = (sr0, sr1)
        sws = (sw0, sw1)

        def fire_r(i):
            return pltpu.async_copy(
                edge_h.at[pl.ds(i * blk, blk)], rbufs[i % 2], srs[i % 2])

        def fire_w(i):
            return pltpu.async_copy(
                wbufs[i % 2],
                o_ref.at[pl.ds(i * blk, blk), pl.ds(d_full, de)],
                sws[i % 2])

        rcps = [None] * n_blk
        wcps = [None] * n_blk
        rcps[0] = fire_r(0)
        for i in range(n_blk):
            if i + 1 < n_blk:
                rcps[i + 1] = fire_r(i + 1)
            rcps[i].wait()
            if i >= 2:
                wcps[i - 2].wait()
            wbufs[i % 2][...] = rbufs[i % 2][:, :de]
            wcps[i] = fire_w(i)
        wcps[n_blk - 2].wait()
        wcps[n_blk - 1].wait()

    return pl.pallas_call(
        body,
        in_specs=[
            pl.BlockSpec(memory_space=pl.ANY),
            pl.BlockSpec(memory_space=pl.ANY),
        ],
        out_specs=pl.BlockSpec(memory_space=pl.ANY),
        out_shape=jax.ShapeDtypeStruct((n, d), jnp.float32),
        scratch_shapes=[
            pltpu.VMEM((blk, 128), jnp.float32),
            pltpu.VMEM((blk, 128), jnp.float32),
            pltpu.VMEM((blk, de), jnp.float32),
            pltpu.VMEM((blk, de), jnp.float32),
            pltpu.SemaphoreType.DMA,
            pltpu.SemaphoreType.DMA,
            pltpu.SemaphoreType.DMA,
            pltpu.SemaphoreType.DMA,
        ],
        input_output_aliases={0: 0},
    )


@functools.cache
def _make_sc_gather(n, v, d, ch, nbuf):
    rows_per = n // NW
    n_chunks = rows_per // ch
    assert rows_per % ch == 0 and n_chunks % nbuf == 0
    mesh = plsc.VectorSubcoreMesh(
        core_axis_name="c", subcore_axis_name="s",
        num_cores=NC, num_subcores=NS,
    )

    row_bufs = [pltpu.VMEM((ch, DPAD), jnp.float32) for _ in range(nbuf)]
    sems = [pltpu.SemaphoreType.DMA for _ in range(2 * nbuf)]

    @functools.partial(
        pl.kernel,
        out_type=(
            jax.ShapeDtypeStruct((n, d), jnp.float32),
            jax.ShapeDtypeStruct((n, 128), jnp.float32),
        ),
        mesh=mesh,
        compiler_params=pltpu.CompilerParams(use_tc_tiling_on_sc=True),
        scratch_types=[pltpu.VMEM((rows_per,), jnp.int32)] + row_bufs + sems,
    )
    def sc_kernel(tpad_h, x_h, out_h, edge_h, x_all, *bufs_and_sems):
        rows = bufs_and_sems[:nbuf]
        sg = bufs_and_sems[nbuf:2 * nbuf]
        ss = bufs_and_sems[2 * nbuf:3 * nbuf]

        cid = lax.axis_index("c")
        sid = lax.axis_index("s")
        wid = sid * NC + cid
        base = wid * rows_per

        pltpu.sync_copy(x_h.at[pl.ds(base, rows_per)], x_all)

        def fire_gather(i, b):
            return pltpu.async_copy(
                tpad_h.at[x_all.at[pl.ds(i * ch, ch)]], rows[b], sg[b])

        def wait_g(b):
            pltpu.make_async_copy(
                tpad_h.at[x_all.at[pl.ds(0, ch)]], rows[b], sg[b]).wait()

        d_full = (d // 128) * 128

        def store(i, b):
            off = base + i * ch
            cps = []
            for cb in range(0, d_full, 128):
                cps.append(pltpu.async_copy(
                    rows[b].at[:, pl.ds(cb, 128)],
                    out_h.at[pl.ds(off, ch), pl.ds(cb, 128)], ss[b]))
            cps.append(pltpu.async_copy(
                rows[b].at[:, pl.ds(d_full, 128)],
                edge_h.at[pl.ds(off, ch)], ss[b]))
            return cps

        for b in range(nbuf):
            fire_gather(b, b)

        def steady(k, carry):
            i0 = k * nbuf
            for b in range(nbuf):
                i = i0 + b
                wait_g(b)
                for cp in store(i, b):
                    cp.wait()
                fire_gather(i + nbuf, b)
            return carry

        lax.fori_loop(0, n_chunks // nbuf - 1, steady, 0)

        i0 = n_chunks - nbuf
        for b in range(nbuf):
            wait_g(b)
            for cp in store(i0 + b, b):
                cp.wait()

    return sc_kernel


def kernel(X, y, table):
    n = X.size
    v, d = table.shape
    blk = 1024
    xf = X.reshape(-1).astype(jnp.int32)
    yf = y.reshape(-1).astype(jnp.int32)
    tpad = jnp.pad(table, ((0, 0), (0, DPAD - d)))
    out1, edge = _make_sc_gather(n, v, d, 16, 5)(tpad, xf)
    x3 = xf.reshape(n // blk, 1, blk)
    raw = _make_loss_tc(n // blk, blk, v)(x3, yf.reshape(n // blk, 1, blk), table)
    out = _make_patch_tc(n, d, 3200)(out1, edge)
    loss = raw[0, 0] / n
    return out, loss
